# baseline (speedup 1.0000x reference)
import jax
import jax.numpy as jnp
from jax.experimental import pallas as pl
import jax.experimental.pallas.tpu as pltpu


_BAND = 128
_CHUNK = 128


def _body(x_ref, o_ref, sem):
    i = pl.program_id(0)
    batch = o_ref.shape[0]
    s = o_ref.shape[2]
    nc = s // _CHUNK

    copies = []
    for k in range(nc):
        sl = slice(k * _CHUNK, (k + 1) * _CHUNK)
        cp = pltpu.make_async_copy(
            x_ref.at[:, pl.ds(i * _BAND, _BAND), sl],
            o_ref.at[:, :, sl],
            sem,
        )
        copies.append(cp)

        @pl.when(k <= i)
        def _(cp=cp):
            cp.start()

    for k in range(nc):
        sl = slice(k * _CHUNK, (k + 1) * _CHUNK)

        @pl.when(k > i)
        def _(sl=sl):
            o_ref[:, :, sl] = jnp.ones((batch, _BAND, _CHUNK), jnp.float32)

    for k in range(nc):
        @pl.when(k <= i)
        def _(cp=copies[k]):
            cp.wait()

    for k in range(nc):
        sl = slice(k * _CHUNK, (k + 1) * _CHUNK)

        @pl.when(k == i)
        def _(sl=sl):
            r = jax.lax.broadcasted_iota(jnp.int32, (1, _BAND, _CHUNK), 1)
            c = jax.lax.broadcasted_iota(jnp.int32, (1, _BAND, _CHUNK), 2)
            o_ref[:, :, sl] = jnp.where(
                c > r, jnp.float32(1.0), o_ref[:, :, sl]
            )


def kernel(x):
    batch, s, _ = x.shape
    n = s // _BAND
    return pl.pallas_call(
        _body,
        grid=(n,),
        in_specs=[pl.BlockSpec(memory_space=pltpu.MemorySpace.HBM)],
        out_specs=pl.BlockSpec((batch, _BAND, s), lambda i: (0, i, 0)),
        out_shape=jax.ShapeDtypeStruct(x.shape, x.dtype),
        scratch_shapes=[pltpu.SemaphoreType.DMA],
    )(x)


# speedup vs baseline: 1.2239x; 1.2239x over previous
import jax
import jax.numpy as jnp
from jax.experimental import pallas as pl
import jax.experimental.pallas.tpu as pltpu


_BAND = 256
_CHUNK = 256


def _body(x_ref, o_ref, sem):
    i = pl.program_id(0)
    batch = o_ref.shape[0]
    s = o_ref.shape[2]
    nc = s // _CHUNK

    copies = []
    for k in range(nc):
        sl = slice(k * _CHUNK, (k + 1) * _CHUNK)
        cp = pltpu.make_async_copy(
            x_ref.at[:, pl.ds(i * _BAND, _BAND), sl],
            o_ref.at[:, :, sl],
            sem,
        )
        copies.append(cp)

        @pl.when(k <= i)
        def _(cp=cp):
            cp.start()

    for k in range(nc):
        sl = slice(k * _CHUNK, (k + 1) * _CHUNK)

        @pl.when(k > i)
        def _(sl=sl):
            o_ref[:, :, sl] = jnp.ones((batch, _BAND, _CHUNK), jnp.float32)

    for k in range(nc):
        @pl.when(k <= i)
        def _(cp=copies[k]):
            cp.wait()

    for k in range(nc):
        sl = slice(k * _CHUNK, (k + 1) * _CHUNK)

        @pl.when(k == i)
        def _(sl=sl):
            r = jax.lax.broadcasted_iota(jnp.int32, (1, _BAND, _CHUNK), 1)
            c = jax.lax.broadcasted_iota(jnp.int32, (1, _BAND, _CHUNK), 2)
            o_ref[:, :, sl] = jnp.where(
                c > r, jnp.float32(1.0), o_ref[:, :, sl]
            )


def kernel(x):
    batch, s, _ = x.shape
    n = s // _BAND
    return pl.pallas_call(
        _body,
        grid=(n,),
        in_specs=[pl.BlockSpec(memory_space=pltpu.MemorySpace.HBM)],
        out_specs=pl.BlockSpec((batch, _BAND, s), lambda i: (0, i, 0)),
        out_shape=jax.ShapeDtypeStruct(x.shape, x.dtype),
        scratch_shapes=[pltpu.SemaphoreType.DMA],
    )(x)


# speedup vs baseline: 1.4631x; 1.1955x over previous
import jax
import jax.numpy as jnp
from jax.experimental import pallas as pl
import jax.experimental.pallas.tpu as pltpu


_BAND = 256
_SLOTS = 3


def _read_band(x_ref, buf, sem_r, band, slot, nc):
    for k in range(nc):
        sl = slice(k * _BAND, (k + 1) * _BAND)

        @pl.when(k <= band)
        def _(sl=sl):
            pltpu.make_async_copy(
                x_ref.at[:, pl.ds(band * _BAND, _BAND), sl],
                buf.at[slot, :, :, sl],
                sem_r.at[slot],
            ).start()


def _wait_band(x_ref, buf, sem_r, band, slot, nc):
    for k in range(nc):
        sl = slice(k * _BAND, (k + 1) * _BAND)

        @pl.when(k <= band)
        def _(sl=sl):
            pltpu.make_async_copy(
                x_ref.at[:, pl.ds(band * _BAND, _BAND), sl],
                buf.at[slot, :, :, sl],
                sem_r.at[slot],
            ).wait()


def _write_copy(o_ref, buf, sem_w, band, slot):
    return pltpu.make_async_copy(
        buf.at[slot],
        o_ref.at[:, pl.ds(band * _BAND, _BAND), :],
        sem_w.at[slot],
    )


def _body(x_ref, o_ref, buf, sem_r, sem_w):
    i = pl.program_id(0)
    n = pl.num_programs(0)
    s = x_ref.shape[2]
    nc = s // _BAND
    slot = jax.lax.rem(i, _SLOTS)
    nxt = jax.lax.rem(i + 1, _SLOTS)

    @pl.when(i == 0)
    def _():
        _read_band(x_ref, buf, sem_r, 0, jnp.int32(0), nc)

    @pl.when(jnp.logical_and(i + 1 < n, i >= _SLOTS - 1))
    def _():
        _write_copy(o_ref, buf, sem_w, i - (_SLOTS - 1), nxt).wait()

    @pl.when(i + 1 < n)
    def _():
        _read_band(x_ref, buf, sem_r, i + 1, nxt, nc)

    for k in range(nc):
        sl = slice(k * _BAND, (k + 1) * _BAND)

        @pl.when(k > i)
        def _(sl=sl):
            buf[slot, :, :, sl] = jnp.ones(
                (buf.shape[1], _BAND, _BAND), jnp.float32
            )

    _wait_band(x_ref, buf, sem_r, i, slot, nc)

    for k in range(nc):
        sl = slice(k * _BAND, (k + 1) * _BAND)

        @pl.when(k == i)
        def _(sl=sl):
            r = jax.lax.broadcasted_iota(jnp.int32, (1, _BAND, _BAND), 1)
            c = jax.lax.broadcasted_iota(jnp.int32, (1, _BAND, _BAND), 2)
            buf[slot, :, :, sl] = jnp.where(
                c > r, jnp.float32(1.0), buf[slot, :, :, sl]
            )

    _write_copy(o_ref, buf, sem_w, i, slot).start()

    n_static = x_ref.shape[1] // _BAND

    @pl.when(i == n - 1)
    def _():
        for band in range(max(0, n_static - _SLOTS), n_static):
            _write_copy(o_ref, buf, sem_w, band, band % _SLOTS).wait()


def kernel(x):
    batch, s, _ = x.shape
    n = s // _BAND
    return pl.pallas_call(
        _body,
        grid=(n,),
        in_specs=[pl.BlockSpec(memory_space=pltpu.MemorySpace.HBM)],
        out_specs=pl.BlockSpec(memory_space=pltpu.MemorySpace.HBM),
        out_shape=jax.ShapeDtypeStruct(x.shape, x.dtype),
        scratch_shapes=[
            pltpu.VMEM((_SLOTS, batch, _BAND, s), jnp.float32),
            pltpu.SemaphoreType.DMA((_SLOTS,)),
            pltpu.SemaphoreType.DMA((_SLOTS,)),
        ],
    )(x)
